# baseline (speedup 1.0000x reference)
import functools

import jax
import jax.numpy as jnp
from jax import lax
from jax.experimental import pallas as pl
from jax.experimental.pallas import tpu as pltpu
from jax.experimental.pallas import tpu_sc as plsc

N = 2048
BATCHES = 4
L = 16
NC = 2
NS = 16
NW = NC * NS
NCHUNK = N // L
G = 8
NGRP = N // G
GRP_PER_W = NGRP // NW


def _sc_body(pos_hbm, meta_hbm, out_hbm,
             posx, posy, posz, metav, dbuf, gbuf, zbig, zsem):
    wid = lax.axis_index("s") * NC + lax.axis_index("c")
    iota = lax.iota(jnp.int32, L)
    zv = jnp.zeros((L,), jnp.float32)
    infv = jnp.full((L,), jnp.inf, jnp.float32)

    def zero_rows(ref, r):
        @plsc.parallel_loop(0, N, L, unroll=4)
        def _zi(off):
            ref[r, pl.ds(off, L)] = zv

    for r in range(G):
        zero_rows(zbig, r)
    pltpu.sync_copy(meta_hbm, metav)

    zcnt = jnp.int32(0)
    for b in range(BATCHES):
        mv = metav[...]
        t = mv[b]
        tau = mv[b + 4]
        n_src = t + tau

        pltpu.sync_copy(pos_hbm.at[pl.ds((b * 3 + 0) * N, N)], posx)
        pltpu.sync_copy(pos_hbm.at[pl.ds((b * 3 + 1) * N, N)], posy)
        pltpu.sync_copy(pos_hbm.at[pl.ds((b * 3 + 2) * N, N)], posz)
        lastc = (n_src // L) * L
        nsv0 = jnp.full((L,), n_src, jnp.int32)
        pois = lastc + iota >= nsv0
        big = jnp.full((L,), 1e30, jnp.float32)
        for ref in (posx, posy, posz):
            ref[pl.ds(lastc, L)] = jnp.where(pois, big,
                                             ref[pl.ds(lastc, L)])

        for k in (1, 2, 3):
            @pl.when(lastc + k * L < N)
            def _pois2(k=k):
                for ref in (posx, posy, posz):
                    ref[pl.ds(lastc + k * L, L)] = big
        for r in range(G):
            zero_rows(gbuf, r)

        cdiv8 = (tau + G - 1) // G
        zg0 = jnp.clip((cdiv8 - wid + NW - 1) // NW, 0, GRP_PER_W)

        def fire_one(st, b=b):
            zi, zf = st
            can = zi < GRP_PER_W

            def yes(_):
                g2 = wid + NW * zi
                goff2 = pl.multiple_of(g2 * G, G)
                pltpu.async_copy(zbig, out_hbm.at[b, pl.ds(goff2, G)],
                                 zsem)
                return 0

            def no(_):
                return 0

            lax.cond(can, yes, no, 0)

            def w(_):
                pltpu.make_async_copy(
                    zbig, out_hbm.at[0, pl.ds(0, G)], zsem).wait()
                return 0

            lax.cond(can & (zf >= 4), w, no, 0)
            inc = can.astype(jnp.int32)
            return (zi + inc, zf + inc)

        def do_group(gi, st):
            g = wid + NW * gi
            goff = pl.multiple_of(g * G, G)

            def comp(st):
                def quad_rows(qi, st):
                    i0 = 4 * qi
                    s0 = goff + i0

                    @pl.when(s0 < tau)
                    def _compute4():
                        sx2, sy2, sz2, sns = [], [], [], []
                        for r in range(4):
                            sidx = jnp.full(
                                (L,),
                                jnp.minimum(t + s0 + r, N - 1),
                                jnp.int32)
                            sx = plsc.load_gather(posx, [sidx])
                            sy = plsc.load_gather(posy, [sidx])
                            sz = plsc.load_gather(posz, [sidx])
                            sx2.append(sx * -2.0)
                            sy2.append(sy * -2.0)
                            sz2.append(sz * -2.0)
                            sns.append((sx * sx + sy * sy) + sz * sz)

                        nq4 = (n_src + 4 * L - 1) // (4 * L)

                        @plsc.parallel_loop(0, nq4 * 4 * L, 4 * L,
                                            unroll=2,
                                            carry=(infv,) * 16)
                        def ks(off0, ks):
                            ks = list(ks)
                            for j in range(4):
                                off = off0 + j * L
                                px = posx[pl.ds(off, L)]
                                py = posy[pl.ds(off, L)]
                                pz = posz[pl.ds(off, L)]
                                pn = (px * px + py * py) + pz * pz
                                for r in range(4):
                                    d = pn + sns[r]
                                    d = px * sx2[r] + d
                                    d = py * sy2[r] + d
                                    d = pz * sz2[r] + d
                                    dbuf[r, pl.ds(off, L)] = d
                                    k = 4 * r + j
                                    dsrt = jnp.sort(d)
                                    ks[k] = jnp.sort(
                                        jnp.minimum(ks[k],
                                                    jnp.flip(dsrt)))
                            return tuple(ks)

                        for r in range(4):
                            s = s0 + r
                            i = i0 + r

                            @pl.when(s < tau)
                            def _row(r=r, s=s, i=i):
                                ka = jnp.sort(
                                    jnp.minimum(ks[4 * r],
                                                jnp.flip(ks[4 * r + 1])))
                                kb = jnp.sort(
                                    jnp.minimum(ks[4 * r + 2],
                                                jnp.flip(ks[4 * r + 3])))
                                keys = jnp.sort(
                                    jnp.minimum(ka, jnp.flip(kb)))
                                thrv = jnp.full((L,), keys[L - 1],
                                                jnp.float32)
                                sv = jnp.full((L,), s, jnp.int32)

                                nw16 = ((s + L - 1) // L) * L

                                @plsc.parallel_loop(0, nw16, L,
                                                    unroll=4)
                                def _wchunk(off, r=r, i=i, thrv=thrv,
                                            sv=sv):
                                    dv = dbuf[r, pl.ds(off, L)]
                                    m = (dv <= thrv) & (off + iota < sv)
                                    gbuf[i, pl.ds(off, L)] = jnp.where(
                                        m, 1.0, 0.0)

                            @pl.when(s >= tau)
                            def _zrow(i=i):
                                zero_rows(gbuf, i)

                    @pl.when(s0 >= tau)
                    def _allz():
                        for r in range(4):
                            zero_rows(gbuf, i0 + r)

                    st = fire_one(st)
                    return fire_one(st)

                st = lax.fori_loop(0, G // 4, quad_rows, st)
                pltpu.sync_copy(gbuf, out_hbm.at[b, pl.ds(goff, G)])
                return st

            def skip(st):
                return st

            return lax.cond(g * G < tau, comp, skip, st)

        st = lax.fori_loop(0, GRP_PER_W, do_group,
                           (zg0, zcnt))

        def tail(i, st):
            return fire_one(st)

        st = lax.fori_loop(0, GRP_PER_W, tail, st)
        zcnt = st[1]

    def drain(i, _):
        pltpu.make_async_copy(zbig, out_hbm.at[0, pl.ds(0, G)], zsem).wait()
        return 0

    lax.fori_loop(0, jnp.minimum(zcnt, 4), drain, 0)


@functools.partial(
    pl.kernel,
    out_type=jax.ShapeDtypeStruct((BATCHES, N, N), jnp.float32),
    mesh=plsc.VectorSubcoreMesh(core_axis_name="c", subcore_axis_name="s",
                                num_cores=NC, num_subcores=NS),
    compiler_params=pltpu.CompilerParams(needs_layout_passes=False),
    scratch_types=[
        pltpu.VMEM((N,), jnp.float32),
        pltpu.VMEM((N,), jnp.float32),
        pltpu.VMEM((N,), jnp.float32),
        pltpu.VMEM((L,), jnp.int32),
        pltpu.VMEM((4, N), jnp.float32),
        pltpu.VMEM((G, N), jnp.float32),
        pltpu.VMEM((G, N), jnp.float32),
        pltpu.SemaphoreType.DMA,
    ],
)
def _sc_knn(pos_hbm, meta_hbm, out_hbm,
            posx, posy, posz, metav, dbuf, gbuf, zbig, zsem):
    _sc_body(pos_hbm, meta_hbm, out_hbm,
             posx, posy, posz, metav, dbuf, gbuf, zbig, zsem)


def kernel(nodes, T, taus, B):
    pos_t = jnp.transpose(nodes[:, :, :3], (0, 2, 1)).reshape(-1)
    meta = jnp.concatenate([T.astype(jnp.int32), taus.astype(jnp.int32),
                            jnp.zeros((8,), jnp.int32)])
    return _sc_knn(pos_t, meta)


# speedup vs baseline: 1.2596x; 1.2596x over previous
import functools

import jax
import jax.numpy as jnp
from jax import lax
from jax.experimental import pallas as pl
from jax.experimental.pallas import tpu as pltpu
from jax.experimental.pallas import tpu_sc as plsc

N = 2048
BATCHES = 4
L = 16
NC = 2
NS = 16
NW = NC * NS
NCHUNK = N // L
G = 8
NGRP = N // G
GRP_PER_W = NGRP // NW


def _sc_body(pos_hbm, meta_hbm, out_hbm,
             posx, posy, posz, metav, dbuf, gbuf, zbig, zsem, gsem):
    wid = lax.axis_index("s") * NC + lax.axis_index("c")
    iota = lax.iota(jnp.int32, L)
    zv = jnp.zeros((L,), jnp.float32)
    infv = jnp.full((L,), jnp.inf, jnp.float32)

    def zero_rows(ref, r):
        @plsc.parallel_loop(0, N, L, unroll=4)
        def _zi(off):
            ref[r, pl.ds(off, L)] = zv

    for r in range(G):
        zero_rows(zbig, r)
    pltpu.sync_copy(meta_hbm, metav)

    zcnt = jnp.int32(0)
    gcnt = jnp.int32(0)
    for b in range(BATCHES):
        mv = metav[...]
        t = mv[b]
        tau = mv[b + 4]
        n_src = t + tau

        pltpu.sync_copy(pos_hbm.at[pl.ds((b * 3 + 0) * N, N)], posx)
        pltpu.sync_copy(pos_hbm.at[pl.ds((b * 3 + 1) * N, N)], posy)
        pltpu.sync_copy(pos_hbm.at[pl.ds((b * 3 + 2) * N, N)], posz)
        lastc = (n_src // L) * L
        nsv0 = jnp.full((L,), n_src, jnp.int32)
        pois = lastc + iota >= nsv0
        big = jnp.full((L,), 1e30, jnp.float32)
        for ref in (posx, posy, posz):
            ref[pl.ds(lastc, L)] = jnp.where(pois, big,
                                             ref[pl.ds(lastc, L)])

        @pl.when(lastc + L < N)
        def _pois2():
            for ref in (posx, posy, posz):
                ref[pl.ds(lastc + L, L)] = big
        def gw0(_):
            pltpu.make_async_copy(gbuf, out_hbm.at[0, pl.ds(0, G)],
                                  gsem).wait()
            return 0

        def gn0(_):
            return 0

        lax.cond(gcnt >= 1, gw0, gn0, 0)
        gcnt = jnp.int32(0)
        for r in range(G):
            zero_rows(gbuf, r)

        cdiv8 = (tau + G - 1) // G
        zg0 = jnp.clip((cdiv8 - wid + NW - 1) // NW, 0, GRP_PER_W)

        def gbuf_wait(gc):
            def w2(_):
                pltpu.make_async_copy(
                    gbuf, out_hbm.at[0, pl.ds(0, G)], gsem).wait()
                return 0

            def n2(_):
                return 0

            lax.cond(gc >= 1, w2, n2, 0)

        def fire_one(st, b=b):
            zi, zf = st
            can = zi < GRP_PER_W

            def yes(_):
                g2 = wid + NW * zi
                goff2 = pl.multiple_of(g2 * G, G)
                pltpu.async_copy(zbig, out_hbm.at[b, pl.ds(goff2, G)],
                                 zsem)
                return 0

            def no(_):
                return 0

            lax.cond(can, yes, no, 0)

            def w(_):
                pltpu.make_async_copy(
                    zbig, out_hbm.at[0, pl.ds(0, G)], zsem).wait()
                return 0

            lax.cond(can & (zf >= 4), w, no, 0)
            inc = can.astype(jnp.int32)
            return (zi + inc, zf + inc)

        def do_group(gi, st):
            g = wid + NW * gi
            goff = pl.multiple_of(g * G, G)

            def comp(st):
                zi0, zf0, gcnt = st
                st = (zi0, zf0)

                def quad_rows(qi, st):
                    i0 = 4 * qi
                    s0 = goff + i0

                    @pl.when(s0 < tau)
                    def _compute4():
                        sx2, sy2, sz2, sns = [], [], [], []
                        for r in range(4):
                            sidx = jnp.full(
                                (L,),
                                jnp.minimum(t + s0 + r, N - 1),
                                jnp.int32)
                            sx = plsc.load_gather(posx, [sidx])
                            sy = plsc.load_gather(posy, [sidx])
                            sz = plsc.load_gather(posz, [sidx])
                            sx2.append(sx * -2.0)
                            sy2.append(sy * -2.0)
                            sz2.append(sz * -2.0)
                            sns.append((sx * sx + sy * sy) + sz * sz)

                        nq2 = (n_src + 2 * L - 1) // (2 * L)

                        @plsc.parallel_loop(0, nq2 * 2 * L, 2 * L,
                                            unroll=2, carry=(infv,) * 8)
                        def ks(off0, ks):
                            ks = list(ks)
                            for j in range(2):
                                off = off0 + j * L
                                px = posx[pl.ds(off, L)]
                                py = posy[pl.ds(off, L)]
                                pz = posz[pl.ds(off, L)]
                                pn = (px * px + py * py) + pz * pz
                                for r in range(4):
                                    d = pn + sns[r]
                                    d = px * sx2[r] + d
                                    d = py * sy2[r] + d
                                    d = pz * sz2[r] + d
                                    dbuf[r, pl.ds(off, L)] = d
                                    k = 2 * r + j
                                    dsrt = jnp.sort(d)
                                    ks[k] = jnp.sort(
                                        jnp.minimum(ks[k],
                                                    jnp.flip(dsrt)))
                            return tuple(ks)

                        @pl.when(qi == 0)
                        def _gw():
                            gbuf_wait(gcnt)

                        for r in range(4):
                            s = s0 + r
                            i = i0 + r

                            @pl.when(s < tau)
                            def _row(r=r, s=s, i=i):
                                keys = jnp.sort(
                                    jnp.minimum(ks[2 * r],
                                                jnp.flip(ks[2 * r + 1])))
                                thrv = jnp.full((L,), keys[L - 1],
                                                jnp.float32)
                                sv = jnp.full((L,), s, jnp.int32)

                                nw16 = ((s + L - 1) // L) * L

                                @plsc.parallel_loop(0, nw16, L,
                                                    unroll=4)
                                def _wchunk(off, r=r, i=i, thrv=thrv,
                                            sv=sv):
                                    dv = dbuf[r, pl.ds(off, L)]
                                    m = (dv <= thrv) & (off + iota < sv)
                                    gbuf[i, pl.ds(off, L)] = jnp.where(
                                        m, 1.0, 0.0)

                            @pl.when(s >= tau)
                            def _zrow(i=i):
                                zero_rows(gbuf, i)

                    @pl.when(s0 >= tau)
                    def _allz():
                        @pl.when(qi == 0)
                        def _gw2():
                            gbuf_wait(gcnt)

                        for r in range(4):
                            zero_rows(gbuf, i0 + r)

                    st = fire_one(st)
                    return fire_one(st)

                st = lax.fori_loop(0, G // 4, quad_rows, st)
                pltpu.async_copy(gbuf, out_hbm.at[b, pl.ds(goff, G)],
                                 gsem)
                return (st[0], st[1], gcnt + 1)

            def skip(st):
                return st

            return lax.cond(g * G < tau, comp, skip, st)

        st = lax.fori_loop(0, GRP_PER_W, do_group,
                           (zg0, zcnt, gcnt))

        def tail(i, st2):
            zi2, zf2 = fire_one((st2[0], st2[1]))
            return (zi2, zf2, st2[2])

        st = lax.fori_loop(0, GRP_PER_W, tail, st)
        zcnt = st[1]
        gcnt = st[2]

    def drain(i, _):
        pltpu.make_async_copy(zbig, out_hbm.at[0, pl.ds(0, G)], zsem).wait()
        return 0

    lax.fori_loop(0, jnp.minimum(zcnt, 4), drain, 0)

    def gdrainf(_):
        pltpu.make_async_copy(gbuf, out_hbm.at[0, pl.ds(0, G)],
                              gsem).wait()
        return 0

    def gnf(_):
        return 0

    lax.cond(gcnt >= 1, gdrainf, gnf, 0)


@functools.partial(
    pl.kernel,
    out_type=jax.ShapeDtypeStruct((BATCHES, N, N), jnp.float32),
    mesh=plsc.VectorSubcoreMesh(core_axis_name="c", subcore_axis_name="s",
                                num_cores=NC, num_subcores=NS),
    compiler_params=pltpu.CompilerParams(needs_layout_passes=False),
    scratch_types=[
        pltpu.VMEM((N,), jnp.float32),
        pltpu.VMEM((N,), jnp.float32),
        pltpu.VMEM((N,), jnp.float32),
        pltpu.VMEM((L,), jnp.int32),
        pltpu.VMEM((4, N), jnp.float32),
        pltpu.VMEM((G, N), jnp.float32),
        pltpu.VMEM((G, N), jnp.float32),
        pltpu.SemaphoreType.DMA,
        pltpu.SemaphoreType.DMA,
    ],
)
def _sc_knn(pos_hbm, meta_hbm, out_hbm,
            posx, posy, posz, metav, dbuf, gbuf, zbig, zsem, gsem):
    _sc_body(pos_hbm, meta_hbm, out_hbm,
             posx, posy, posz, metav, dbuf, gbuf, zbig, zsem, gsem)


def kernel(nodes, T, taus, B):
    pos_t = jnp.transpose(nodes[:, :, :3], (0, 2, 1)).reshape(-1)
    meta = jnp.concatenate([T.astype(jnp.int32), taus.astype(jnp.int32),
                            jnp.zeros((8,), jnp.int32)])
    return _sc_knn(pos_t, meta)


# speedup vs baseline: 1.2614x; 1.0014x over previous
import functools

import jax
import jax.numpy as jnp
from jax import lax
from jax.experimental import pallas as pl
from jax.experimental.pallas import tpu as pltpu
from jax.experimental.pallas import tpu_sc as plsc

N = 2048
BATCHES = 4
L = 16
NC = 2
NS = 16
NW = NC * NS
NCHUNK = N // L
G = 8
NGRP = N // G
GRP_PER_W = NGRP // NW


def _sc_body(pos_hbm, meta_hbm, out_hbm,
             posx, posy, posz, metav, dbuf, gbuf, zbig, zsem, gsem):
    wid = lax.axis_index("s") * NC + lax.axis_index("c")
    iota = lax.iota(jnp.int32, L)
    zv = jnp.zeros((L,), jnp.float32)
    infv = jnp.full((L,), jnp.inf, jnp.float32)

    def zero_rows(ref, r):
        @plsc.parallel_loop(0, N, L, unroll=4)
        def _zi(off):
            ref[r, pl.ds(off, L)] = zv

    for r in range(G):
        zero_rows(zbig, r)
    pltpu.sync_copy(meta_hbm, metav)

    zcnt = jnp.int32(0)
    gcnt = jnp.int32(0)
    for b in range(BATCHES):
        mv = metav[...]
        t = mv[b]
        tau = mv[b + 4]
        n_src = t + tau

        pltpu.sync_copy(pos_hbm.at[pl.ds((b * 3 + 0) * N, N)], posx)
        pltpu.sync_copy(pos_hbm.at[pl.ds((b * 3 + 1) * N, N)], posy)
        pltpu.sync_copy(pos_hbm.at[pl.ds((b * 3 + 2) * N, N)], posz)
        lastc = (n_src // L) * L
        nsv0 = jnp.full((L,), n_src, jnp.int32)
        pois = lastc + iota >= nsv0
        big = jnp.full((L,), 1e30, jnp.float32)
        for ref in (posx, posy, posz):
            ref[pl.ds(lastc, L)] = jnp.where(pois, big,
                                             ref[pl.ds(lastc, L)])

        @pl.when(lastc + L < N)
        def _pois2():
            for ref in (posx, posy, posz):
                ref[pl.ds(lastc + L, L)] = big
        def gw0(_):
            pltpu.make_async_copy(gbuf, out_hbm.at[0, pl.ds(0, G)],
                                  gsem).wait()
            return 0

        def gn0(_):
            return 0

        lax.cond(gcnt >= 1, gw0, gn0, 0)
        gcnt = jnp.int32(0)
        for r in range(G):
            zero_rows(gbuf, r)

        cdiv8 = (tau + G - 1) // G
        zg0 = jnp.clip((cdiv8 - wid + NW - 1) // NW, 0, GRP_PER_W)

        def gbuf_wait(gc):
            def w2(_):
                pltpu.make_async_copy(
                    gbuf, out_hbm.at[0, pl.ds(0, G)], gsem).wait()
                return 0

            def n2(_):
                return 0

            lax.cond(gc >= 1, w2, n2, 0)

        def fire_one(st, b=b):
            zi, zf = st
            can = zi < GRP_PER_W

            def yes(_):
                g2 = wid + NW * zi
                goff2 = pl.multiple_of(g2 * G, G)
                pltpu.async_copy(zbig, out_hbm.at[b, pl.ds(goff2, G)],
                                 zsem)
                return 0

            def no(_):
                return 0

            lax.cond(can, yes, no, 0)

            def w(_):
                pltpu.make_async_copy(
                    zbig, out_hbm.at[0, pl.ds(0, G)], zsem).wait()
                return 0

            lax.cond(can & (zf >= 8), w, no, 0)
            inc = can.astype(jnp.int32)
            return (zi + inc, zf + inc)

        def do_group(gi, st):
            g = wid + NW * gi
            goff = pl.multiple_of(g * G, G)

            def comp(st):
                zi0, zf0, gcnt = st
                st = (zi0, zf0)

                def quad_rows(qi, st):
                    i0 = 4 * qi
                    s0 = goff + i0

                    @pl.when(s0 < tau)
                    def _compute4():
                        sx2, sy2, sz2, sns = [], [], [], []
                        for r in range(4):
                            sidx = jnp.full(
                                (L,),
                                jnp.minimum(t + s0 + r, N - 1),
                                jnp.int32)
                            sx = plsc.load_gather(posx, [sidx])
                            sy = plsc.load_gather(posy, [sidx])
                            sz = plsc.load_gather(posz, [sidx])
                            sx2.append(sx * -2.0)
                            sy2.append(sy * -2.0)
                            sz2.append(sz * -2.0)
                            sns.append((sx * sx + sy * sy) + sz * sz)

                        nq2 = (n_src + 2 * L - 1) // (2 * L)

                        @plsc.parallel_loop(0, nq2 * 2 * L, 2 * L,
                                            unroll=2, carry=(infv,) * 8)
                        def ks(off0, ks):
                            ks = list(ks)
                            for j in range(2):
                                off = off0 + j * L
                                px = posx[pl.ds(off, L)]
                                py = posy[pl.ds(off, L)]
                                pz = posz[pl.ds(off, L)]
                                pn = (px * px + py * py) + pz * pz
                                for r in range(4):
                                    d = pn + sns[r]
                                    d = px * sx2[r] + d
                                    d = py * sy2[r] + d
                                    d = pz * sz2[r] + d
                                    dbuf[r, pl.ds(off, L)] = d
                                    k = 2 * r + j
                                    dsrt = jnp.sort(d)
                                    ks[k] = jnp.sort(
                                        jnp.minimum(ks[k],
                                                    jnp.flip(dsrt)))
                            return tuple(ks)

                        @pl.when(qi == 0)
                        def _gw():
                            gbuf_wait(gcnt)

                        for r in range(4):
                            s = s0 + r
                            i = i0 + r

                            @pl.when(s < tau)
                            def _row(r=r, s=s, i=i):
                                keys = jnp.sort(
                                    jnp.minimum(ks[2 * r],
                                                jnp.flip(ks[2 * r + 1])))
                                thrv = jnp.full((L,), keys[L - 1],
                                                jnp.float32)
                                sv = jnp.full((L,), s, jnp.int32)

                                nw16 = ((s + L - 1) // L) * L

                                @plsc.parallel_loop(0, nw16, L,
                                                    unroll=4)
                                def _wchunk(off, r=r, i=i, thrv=thrv,
                                            sv=sv):
                                    dv = dbuf[r, pl.ds(off, L)]
                                    m = (dv <= thrv) & (off + iota < sv)
                                    gbuf[i, pl.ds(off, L)] = jnp.where(
                                        m, 1.0, 0.0)

                            @pl.when(s >= tau)
                            def _zrow(i=i):
                                zero_rows(gbuf, i)

                    @pl.when(s0 >= tau)
                    def _allz():
                        @pl.when(qi == 0)
                        def _gw2():
                            gbuf_wait(gcnt)

                        for r in range(4):
                            zero_rows(gbuf, i0 + r)

                    st = fire_one(st)
                    return fire_one(st)

                st = lax.fori_loop(0, G // 4, quad_rows, st)
                pltpu.async_copy(gbuf, out_hbm.at[b, pl.ds(goff, G)],
                                 gsem)
                return (st[0], st[1], gcnt + 1)

            def skip(st):
                return st

            return lax.cond(g * G < tau, comp, skip, st)

        st = lax.fori_loop(0, GRP_PER_W, do_group,
                           (zg0, zcnt, gcnt))

        def tail(i, st2):
            zi2, zf2 = fire_one((st2[0], st2[1]))
            return (zi2, zf2, st2[2])

        st = lax.fori_loop(0, GRP_PER_W, tail, st)
        zcnt = st[1]
        gcnt = st[2]

    def drain(i, _):
        pltpu.make_async_copy(zbig, out_hbm.at[0, pl.ds(0, G)], zsem).wait()
        return 0

    lax.fori_loop(0, jnp.minimum(zcnt, 8), drain, 0)

    def gdrainf(_):
        pltpu.make_async_copy(gbuf, out_hbm.at[0, pl.ds(0, G)],
                              gsem).wait()
        return 0

    def gnf(_):
        return 0

    lax.cond(gcnt >= 1, gdrainf, gnf, 0)


@functools.partial(
    pl.kernel,
    out_type=jax.ShapeDtypeStruct((BATCHES, N, N), jnp.float32),
    mesh=plsc.VectorSubcoreMesh(core_axis_name="c", subcore_axis_name="s",
                                num_cores=NC, num_subcores=NS),
    compiler_params=pltpu.CompilerParams(needs_layout_passes=False),
    scratch_types=[
        pltpu.VMEM((N,), jnp.float32),
        pltpu.VMEM((N,), jnp.float32),
        pltpu.VMEM((N,), jnp.float32),
        pltpu.VMEM((L,), jnp.int32),
        pltpu.VMEM((4, N), jnp.float32),
        pltpu.VMEM((G, N), jnp.float32),
        pltpu.VMEM((G, N), jnp.float32),
        pltpu.SemaphoreType.DMA,
        pltpu.SemaphoreType.DMA,
    ],
)
def _sc_knn(pos_hbm, meta_hbm, out_hbm,
            posx, posy, posz, metav, dbuf, gbuf, zbig, zsem, gsem):
    _sc_body(pos_hbm, meta_hbm, out_hbm,
             posx, posy, posz, metav, dbuf, gbuf, zbig, zsem, gsem)


def kernel(nodes, T, taus, B):
    pos_t = jnp.transpose(nodes[:, :, :3], (0, 2, 1)).reshape(-1)
    meta = jnp.concatenate([T.astype(jnp.int32), taus.astype(jnp.int32),
                            jnp.zeros((8,), jnp.int32)])
    return _sc_knn(pos_t, meta)


# speedup vs baseline: 1.3211x; 1.0473x over previous
import functools

import jax
import jax.numpy as jnp
from jax import lax
from jax.experimental import pallas as pl
from jax.experimental.pallas import tpu as pltpu
from jax.experimental.pallas import tpu_sc as plsc

N = 2048
BATCHES = 4
L = 16
NC = 2
NS = 16
NW = NC * NS
NCHUNK = N // L
G = 8
NGRP = N // G
GRP_PER_W = NGRP // NW


def _sc_body(pos_hbm, meta_hbm, out_hbm,
             posx, posy, posz, metav, dbuf, gbuf, zbig, zsem, gsem):
    wid = lax.axis_index("s") * NC + lax.axis_index("c")
    iota = lax.iota(jnp.int32, L)
    zv = jnp.zeros((L,), jnp.float32)
    infv = jnp.full((L,), jnp.inf, jnp.float32)

    def zero_rows(ref, r):
        @plsc.parallel_loop(0, N, L, unroll=4)
        def _zi(off):
            ref[r, pl.ds(off, L)] = zv

    for r in range(G):
        zero_rows(zbig, r)
    pltpu.sync_copy(meta_hbm, metav)

    zcnt = jnp.int32(0)
    gcnt = jnp.int32(0)
    for b in range(BATCHES):
        mv = metav[...]
        t = mv[b]
        tau = mv[b + 4]
        n_src = t + tau

        pltpu.sync_copy(pos_hbm.at[pl.ds((b * 3 + 0) * N, N)], posx)
        pltpu.sync_copy(pos_hbm.at[pl.ds((b * 3 + 1) * N, N)], posy)
        pltpu.sync_copy(pos_hbm.at[pl.ds((b * 3 + 2) * N, N)], posz)
        lastc = (n_src // L) * L
        nsv0 = jnp.full((L,), n_src, jnp.int32)
        pois = lastc + iota >= nsv0
        big = jnp.full((L,), 1e30, jnp.float32)
        for ref in (posx, posy, posz):
            ref[pl.ds(lastc, L)] = jnp.where(pois, big,
                                             ref[pl.ds(lastc, L)])

        @pl.when(lastc + L < N)
        def _pois2():
            for ref in (posx, posy, posz):
                ref[pl.ds(lastc + L, L)] = big
        def gw0(_):
            pltpu.make_async_copy(gbuf, out_hbm.at[0, pl.ds(0, G)],
                                  gsem).wait()
            return 0

        def gn0(_):
            return 0

        lax.cond(gcnt >= 1, gw0, gn0, 0)
        gcnt = jnp.int32(0)
        for r in range(G):
            zero_rows(gbuf, r)

        cdiv8 = (tau + G - 1) // G
        zg0 = jnp.clip((cdiv8 - wid + NW - 1) // NW, 0, GRP_PER_W)

        def gbuf_wait(gc):
            def w2(_):
                pltpu.make_async_copy(
                    gbuf, out_hbm.at[0, pl.ds(0, G)], gsem).wait()
                return 0

            def n2(_):
                return 0

            lax.cond(gc >= 1, w2, n2, 0)

        def fire_one(st, b=b):
            zi, zf = st
            can = zi < GRP_PER_W

            def yes(_):
                g2 = wid + NW * zi
                goff2 = pl.multiple_of(g2 * G, G)
                pltpu.async_copy(zbig, out_hbm.at[b, pl.ds(goff2, G)],
                                 zsem)
                return 0

            def no(_):
                return 0

            lax.cond(can, yes, no, 0)

            def w(_):
                pltpu.make_async_copy(
                    zbig, out_hbm.at[0, pl.ds(0, G)], zsem).wait()
                return 0

            lax.cond(can & (zf >= 8), w, no, 0)
            inc = can.astype(jnp.int32)
            return (zi + inc, zf + inc)

        def do_group(gi, st):
            g = wid + NW * gi
            goff = pl.multiple_of(g * G, G)

            def comp(st):
                zi0, zf0, gcnt = st
                st = (zi0, zf0)

                def quad_rows(qi, st):
                    i0 = 4 * qi
                    s0 = goff + i0

                    @pl.when(s0 < tau)
                    def _compute4():
                        sx2, sy2, sz2 = [], [], []
                        for r in range(4):
                            sidx = jnp.full(
                                (L,),
                                jnp.minimum(t + s0 + r, N - 1),
                                jnp.int32)
                            sx2.append(plsc.load_gather(posx, [sidx])
                                       * -2.0)
                            sy2.append(plsc.load_gather(posy, [sidx])
                                       * -2.0)
                            sz2.append(plsc.load_gather(posz, [sidx])
                                       * -2.0)

                        nq2 = (n_src + 2 * L - 1) // (2 * L)

                        @plsc.parallel_loop(0, nq2 * 2 * L, 2 * L,
                                            unroll=2, carry=(infv,) * 8)
                        def ks(off0, ks):
                            ks = list(ks)
                            for j in range(2):
                                off = off0 + j * L
                                px = posx[pl.ds(off, L)]
                                py = posy[pl.ds(off, L)]
                                pz = posz[pl.ds(off, L)]
                                pn = (px * px + py * py) + pz * pz
                                for r in range(4):
                                    d = px * sx2[r] + pn
                                    d = py * sy2[r] + d
                                    d = pz * sz2[r] + d
                                    dbuf[r, pl.ds(off, L)] = d
                                    k = 2 * r + j
                                    dsrt = jnp.sort(d)
                                    ks[k] = jnp.sort(
                                        jnp.minimum(ks[k],
                                                    jnp.flip(dsrt)))
                            return tuple(ks)

                        @pl.when(qi == 0)
                        def _gw():
                            gbuf_wait(gcnt)

                        for r in range(4):
                            s = s0 + r
                            i = i0 + r

                            @pl.when(s < tau)
                            def _row(r=r, s=s, i=i):
                                keys = jnp.sort(
                                    jnp.minimum(ks[2 * r],
                                                jnp.flip(ks[2 * r + 1])))
                                thrv = jnp.full((L,), keys[L - 1],
                                                jnp.float32)
                                sv = jnp.full((L,), s, jnp.int32)

                                nw16 = ((s + L - 1) // L) * L

                                @plsc.parallel_loop(0, nw16, L,
                                                    unroll=4)
                                def _wchunk(off, r=r, i=i, thrv=thrv,
                                            sv=sv):
                                    dv = dbuf[r, pl.ds(off, L)]
                                    m = (dv <= thrv) & (off + iota < sv)
                                    gbuf[i, pl.ds(off, L)] = jnp.where(
                                        m, 1.0, 0.0)

                            @pl.when(s >= tau)
                            def _zrow(i=i):
                                zero_rows(gbuf, i)

                    @pl.when(s0 >= tau)
                    def _allz():
                        @pl.when(qi == 0)
                        def _gw2():
                            gbuf_wait(gcnt)

                        for r in range(4):
                            zero_rows(gbuf, i0 + r)

                    st = fire_one(st)
                    return fire_one(st)

                st = lax.fori_loop(0, G // 4, quad_rows, st)
                pltpu.async_copy(gbuf, out_hbm.at[b, pl.ds(goff, G)],
                                 gsem)
                return (st[0], st[1], gcnt + 1)

            def skip(st):
                return st

            return lax.cond(g * G < tau, comp, skip, st)

        st = lax.fori_loop(0, GRP_PER_W, do_group,
                           (zg0, zcnt, gcnt))

        def tail(i, st2):
            zi2, zf2 = fire_one((st2[0], st2[1]))
            return (zi2, zf2, st2[2])

        st = lax.fori_loop(0, GRP_PER_W, tail, st)
        zcnt = st[1]
        gcnt = st[2]

    def drain(i, _):
        pltpu.make_async_copy(zbig, out_hbm.at[0, pl.ds(0, G)], zsem).wait()
        return 0

    lax.fori_loop(0, jnp.minimum(zcnt, 8), drain, 0)

    def gdrainf(_):
        pltpu.make_async_copy(gbuf, out_hbm.at[0, pl.ds(0, G)],
                              gsem).wait()
        return 0

    def gnf(_):
        return 0

    lax.cond(gcnt >= 1, gdrainf, gnf, 0)


@functools.partial(
    pl.kernel,
    out_type=jax.ShapeDtypeStruct((BATCHES, N, N), jnp.float32),
    mesh=plsc.VectorSubcoreMesh(core_axis_name="c", subcore_axis_name="s",
                                num_cores=NC, num_subcores=NS),
    compiler_params=pltpu.CompilerParams(needs_layout_passes=False),
    scratch_types=[
        pltpu.VMEM((N,), jnp.float32),
        pltpu.VMEM((N,), jnp.float32),
        pltpu.VMEM((N,), jnp.float32),
        pltpu.VMEM((L,), jnp.int32),
        pltpu.VMEM((4, N), jnp.float32),
        pltpu.VMEM((G, N), jnp.float32),
        pltpu.VMEM((G, N), jnp.float32),
        pltpu.SemaphoreType.DMA,
        pltpu.SemaphoreType.DMA,
    ],
)
def _sc_knn(pos_hbm, meta_hbm, out_hbm,
            posx, posy, posz, metav, dbuf, gbuf, zbig, zsem, gsem):
    _sc_body(pos_hbm, meta_hbm, out_hbm,
             posx, posy, posz, metav, dbuf, gbuf, zbig, zsem, gsem)


def kernel(nodes, T, taus, B):
    pos_t = jnp.transpose(nodes[:, :, :3], (0, 2, 1)).reshape(-1)
    meta = jnp.concatenate([T.astype(jnp.int32), taus.astype(jnp.int32),
                            jnp.zeros((8,), jnp.int32)])
    return _sc_knn(pos_t, meta)
